# Initial kernel scaffold; baseline (speedup 1.0000x reference)
import jax
import jax.numpy as jnp
from jax.experimental import pallas as pl


def kernel(encoded_action, values_var):
    raise NotImplementedError("write your pallas kernel here")


# speedup vs baseline: 1.0574x; 1.0574x over previous
import math

import jax
import jax.numpy as jnp
from jax.experimental import pallas as pl
import jax.experimental.pallas.tpu as pltpu

MEM = 100000
D = 64
B = 128
BS = 5000
NB = MEM // BS
INV_TAU = 1.0 / (0.11 - math.log10(float(MEM)) * 0.01)


def _flash_body(q_ref, v_ref, o_ref, acc_ref, l_ref):
    i = pl.program_id(0)
    q = q_ref[...]
    n = jnp.sqrt(jnp.sum(q * q, axis=1, keepdims=True))
    qn = q / jnp.maximum(n, 1e-12)
    v = v_ref[...]
    s = jax.lax.dot_general(
        qn, v, (((1,), (1,)), ((), ())), preferred_element_type=jnp.float32
    )
    w = jnp.exp(s * INV_TAU)
    lsum = jnp.sum(w, axis=1, keepdims=True)
    contrib = jax.lax.dot_general(
        w, v, (((1,), (0,)), ((), ())), preferred_element_type=jnp.float32
    )

    @pl.when(i == 0)
    def _():
        acc_ref[...] = contrib
        l_ref[...] = lsum

    @pl.when(i > 0)
    def _():
        acc_ref[...] += contrib
        l_ref[...] += lsum

    @pl.when(i == NB - 1)
    def _():
        o_ref[...] = acc_ref[...] / l_ref[...]


def kernel(encoded_action, values_var):
    return pl.pallas_call(
        _flash_body,
        grid=(NB,),
        in_specs=[
            pl.BlockSpec((B, D), lambda i: (0, 0)),
            pl.BlockSpec((BS, D), lambda i: (i, 0)),
        ],
        out_specs=pl.BlockSpec((B, D), lambda i: (0, 0)),
        out_shape=jax.ShapeDtypeStruct((B, D), jnp.float32),
        scratch_shapes=[
            pltpu.VMEM((B, D), jnp.float32),
            pltpu.VMEM((B, 1), jnp.float32),
        ],
        compiler_params=pltpu.CompilerParams(
            dimension_semantics=("arbitrary",),
        ),
    )(encoded_action, values_var)
